# baseline (speedup 1.0000x reference)
import jax
import jax.numpy as jnp
from jax.experimental import pallas as pl
from jax.experimental.pallas import tpu as pltpu

_F = 26
_NB = 10000
_BATCH = 16384
_BSL = 2048


def _body(feats_ref, base_ref, ids_ref, mask_ref):
    x = feats_ref[...]
    r = x
    for c in (8 * _NB, 4 * _NB, 2 * _NB, _NB):
        r = jnp.where(r >= c, r - c, r)
    valid = x > 0
    vid = jnp.where(valid, base_ref[:, 0:1] + r + 1, 0)
    ids_ref[...] = vid
    mask_ref[...] = valid


def _tokenize(feats_t, base_col):
    return pl.pallas_call(
        _body,
        grid=(_BATCH // _BSL,),
        in_specs=[
            pl.BlockSpec((_F, _BSL), lambda i: (0, i)),
            pl.BlockSpec((_F, 128), lambda i: (0, 0)),
        ],
        out_specs=[
            pl.BlockSpec((_F, _BSL), lambda i: (0, i)),
            pl.BlockSpec((_F, _BSL), lambda i: (0, i)),
        ],
        out_shape=[
            jax.ShapeDtypeStruct((_F, _BATCH), jnp.int32),
            jax.ShapeDtypeStruct((_F, _BATCH), jnp.bool_),
        ],
        compiler_params=pltpu.CompilerParams(
            dimension_semantics=("parallel",),
            allow_input_fusion=[True, True],
        ),
    )(feats_t, base_col)


def kernel(int_feats, col_offsets, id_bases):
    del col_offsets
    feats_t = int_feats.T
    base_col = jnp.broadcast_to(id_bases[:, None], (_F, 128))
    ids_t, mask_t = _tokenize(feats_t, base_col)
    return ids_t.T, mask_t.T


# speedup vs baseline: 1.1816x; 1.1816x over previous
import jax
import jax.numpy as jnp
from jax.experimental import pallas as pl
from jax.experimental.pallas import tpu as pltpu

_F = 26
_NB = 10000
_BATCH = 16384
_BSL = 4096


def _body(feats_ref, base_ref, ids_ref, mask_ref):
    x = feats_ref[...]
    r = x
    for c in (8 * _NB, 4 * _NB, 2 * _NB, _NB):
        r = jnp.where(r >= c, r - c, r)
    valid = x > 0
    vid = jnp.where(valid, base_ref[:, 0:1] + r + 1, 0)
    ids_ref[...] = vid
    mask_ref[...] = valid


def _tokenize(feats_t, base_col):
    return pl.pallas_call(
        _body,
        grid=(_BATCH // _BSL,),
        in_specs=[
            pl.BlockSpec((_F, _BSL), lambda i: (0, i)),
            pl.BlockSpec((_F, 128), lambda i: (0, 0)),
        ],
        out_specs=[
            pl.BlockSpec((_F, _BSL), lambda i: (0, i)),
            pl.BlockSpec((_F, _BSL), lambda i: (0, i)),
        ],
        out_shape=[
            jax.ShapeDtypeStruct((_F, _BATCH), jnp.int32),
            jax.ShapeDtypeStruct((_F, _BATCH), jnp.bool_),
        ],
        compiler_params=pltpu.CompilerParams(
            dimension_semantics=("parallel",),
            allow_input_fusion=[True, False],
        ),
    )(feats_t, base_col)


def kernel(int_feats, col_offsets, id_bases):
    del col_offsets
    feats_t = int_feats.T
    base_col = jnp.broadcast_to(id_bases[:, None], (_F, 128))
    ids_t, mask_t = _tokenize(feats_t, base_col)
    return ids_t.T, mask_t.T
